# baseline (speedup 1.0000x reference)
import jax
import jax.numpy as jnp
from jax.experimental import pallas as pl
from jax.experimental.pallas import tpu as pltpu
from jax.experimental.pallas import tpu_sc as plsc

WINDOW = 1024


def kernel(predicate_indices, table):
    batch, hist = predicate_indices.shape
    num_idx = batch * hist
    embed_dim = table.shape[1]
    idx = predicate_indices.reshape(num_idx).astype(jnp.int32)

    mesh = plsc.VectorSubcoreMesh(core_axis_name="c", subcore_axis_name="s")

    @pl.kernel(
        out_type=jax.ShapeDtypeStruct((num_idx, embed_dim), table.dtype),
        mesh=mesh,
        compiler_params=pltpu.CompilerParams(use_tc_tiling_on_sc=False),
    )
    def gather_kernel(table_hbm, idx_hbm, out_hbm):
        def body(idx_vmem, out_vmem):
            pltpu.sync_copy(table_hbm.at[idx_vmem], out_vmem)

        pltpu.emit_pipeline(
            body,
            grid=(num_idx // WINDOW,),
            in_specs=[pl.BlockSpec((WINDOW,), index_map=lambda i: (i,))],
            out_specs=[
                pl.BlockSpec((WINDOW, embed_dim), index_map=lambda i: (i, 0))
            ],
            core_axis_name=("c", "s"),
            dimension_semantics=(pltpu.PARALLEL,),
        )(idx_hbm, out_hbm)

    out = gather_kernel(table, idx)
    return out.reshape(batch, hist, embed_dim)


# speedup vs baseline: 1.6140x; 1.6140x over previous
import jax
import jax.numpy as jnp
from jax.experimental import pallas as pl
from jax.experimental.pallas import tpu as pltpu
from jax.experimental.pallas import tpu_sc as plsc

ROWS = 16


def kernel(predicate_indices, table):
    batch, hist = predicate_indices.shape
    embed_dim = table.shape[1]
    idx = predicate_indices.astype(jnp.int32)

    mesh = plsc.VectorSubcoreMesh(core_axis_name="c", subcore_axis_name="s")

    @pl.kernel(
        out_type=jax.ShapeDtypeStruct((batch, hist, embed_dim), table.dtype),
        mesh=mesh,
        scratch_types=[pltpu.SemaphoreType.DMA],
        compiler_params=pltpu.CompilerParams(use_tc_tiling_on_sc=False),
    )
    def gather_kernel(table_hbm, idx_hbm, out_hbm, sem):
        def body(idx_vmem, out_vmem):
            copies = [
                pltpu.async_copy(
                    table_hbm.at[idx_vmem.at[r]], out_vmem.at[r], sem
                )
                for r in range(ROWS)
            ]
            for c in copies:
                c.wait()

        pltpu.emit_pipeline(
            body,
            grid=(batch // ROWS,),
            in_specs=[pl.BlockSpec((ROWS, hist), index_map=lambda i: (i, 0))],
            out_specs=[
                pl.BlockSpec(
                    (ROWS, hist, embed_dim), index_map=lambda i: (i, 0, 0)
                )
            ],
            core_axis_name=("c", "s"),
            dimension_semantics=(pltpu.PARALLEL,),
        )(idx_hbm, out_hbm)

    return gather_kernel(table, idx)


# speedup vs baseline: 1.6186x; 1.0029x over previous
import jax
import jax.numpy as jnp
from jax.experimental import pallas as pl
from jax.experimental.pallas import tpu as pltpu
from jax.experimental.pallas import tpu_sc as plsc

ROWS = 32


def kernel(predicate_indices, table):
    batch, hist = predicate_indices.shape
    embed_dim = table.shape[1]
    idx = predicate_indices.astype(jnp.int32)

    mesh = plsc.VectorSubcoreMesh(core_axis_name="c", subcore_axis_name="s")

    @pl.kernel(
        out_type=jax.ShapeDtypeStruct((batch, hist, embed_dim), table.dtype),
        mesh=mesh,
        scratch_types=[pltpu.SemaphoreType.DMA],
        compiler_params=pltpu.CompilerParams(use_tc_tiling_on_sc=False),
    )
    def gather_kernel(table_hbm, idx_hbm, out_hbm, sem):
        def body(idx_vmem, out_vmem):
            copies = [
                pltpu.async_copy(
                    table_hbm.at[idx_vmem.at[r]], out_vmem.at[r], sem
                )
                for r in range(ROWS)
            ]
            for c in copies:
                c.wait()

        pltpu.emit_pipeline(
            body,
            grid=(batch // ROWS,),
            in_specs=[pl.BlockSpec((ROWS, hist), index_map=lambda i: (i, 0))],
            out_specs=[
                pl.BlockSpec(
                    (ROWS, hist, embed_dim), index_map=lambda i: (i, 0, 0)
                )
            ],
            core_axis_name=("c", "s"),
            dimension_semantics=(pltpu.PARALLEL,),
        )(idx_hbm, out_hbm)

    return gather_kernel(table, idx)
